# baseline (speedup 1.0000x reference)
import jax
import jax.numpy as jnp
from jax import lax
from jax.experimental import pallas as pl
from jax.experimental.pallas import tpu as pltpu

_N = 10000
_E = 160000
_D = 256
_H = 256
_R = 8
_NB = 400
_GRID = _N // _NB


def _wflat_body(wcomp_ref, bases_ref, out_ref):
    for r in range(_R):
        w = wcomp_ref[r, 0] * bases_ref[0]
        for b in range(1, 4):
            w = w + wcomp_ref[r, b] * bases_ref[b]
        out_ref[r * _D:(r + 1) * _D, :] = w


def _build_wflat(wcomp, bases):
    return pl.pallas_call(
        _wflat_body,
        out_shape=jax.ShapeDtypeStruct((_R * _D, _H), jnp.float32),
        in_specs=[
            pl.BlockSpec(memory_space=pltpu.SMEM),
            pl.BlockSpec(memory_space=pltpu.VMEM),
        ],
        out_specs=pl.BlockSpec(memory_space=pltpu.VMEM),
    )(wcomp, bases)


def _tc_body(agg_ref, deg_ref, h_ref, w_ref, loop_ref, bias_ref, out_ref):
    rec = 1.0 / jnp.maximum(deg_ref[...], 1.0)
    z = jnp.dot(h_ref[...], loop_ref[...],
                preferred_element_type=jnp.float32,
                precision=lax.Precision.HIGHEST)
    for r in range(_R):
        a = agg_ref[:, r * _D:(r + 1) * _D] * rec[:, r:r + 1]
        z = z + jnp.dot(a, w_ref[r * _D:(r + 1) * _D, :],
                        preferred_element_type=jnp.float32,
                        precision=lax.Precision.HIGHEST)
    z = z + bias_ref[0:1, :]
    out_ref[...] = jnp.maximum(z, 0.0)


def _tc_layer(agg, deg, h, w_flat, loop_w, bias8):
    return pl.pallas_call(
        _tc_body,
        grid=(_GRID,),
        out_shape=jax.ShapeDtypeStruct((_N, _H), jnp.float32),
        in_specs=[
            pl.BlockSpec((_NB, _R * _D), lambda i: (i, 0)),
            pl.BlockSpec((_NB, _R), lambda i: (i, 0)),
            pl.BlockSpec((_NB, _D), lambda i: (i, 0)),
            pl.BlockSpec((_R * _D, _H), lambda i: (0, 0)),
            pl.BlockSpec((_D, _H), lambda i: (0, 0)),
            pl.BlockSpec((8, _H), lambda i: (0, 0)),
        ],
        out_specs=pl.BlockSpec((_NB, _H), lambda i: (i, 0)),
    )(agg, deg, h, w_flat, loop_w, bias8)


def kernel(x, edge_index, edge_type, bases1, wcomp1, loop1, bias1,
           bases2, wcomp2, loop2, bias2):
    src = edge_index[0]
    dst = edge_index[1]
    seg = dst * _R + edge_type
    deg = jax.ops.segment_sum(jnp.ones((_E,), jnp.float32), seg,
                              num_segments=_N * _R).reshape(_N, _R)

    h = x
    for bases, wcomp, loop_w, bias in (
            (bases1, wcomp1, loop1, bias1), (bases2, wcomp2, loop2, bias2)):
        agg = jax.ops.segment_sum(h[src], seg,
                                  num_segments=_N * _R).reshape(_N, _R * _D)
        w_flat = _build_wflat(wcomp, bases)
        h = _tc_layer(agg, deg, h, w_flat, loop_w,
                      jnp.broadcast_to(bias.reshape(1, _H), (8, _H)))
    return h
